# baseline (speedup 1.0000x reference)
import jax
import jax.numpy as jnp
from jax import lax
from jax.experimental import pallas as pl
from jax.experimental.pallas import tpu as pltpu
from jax.experimental.pallas import tpu_sc as plsc

NC = 2
NS = 16
NW = NC * NS

CHUNK = 64
N_PAD = 10112
ROWS_PER_TILE = N_PAD // NS

NBUF = 4
PHASE = 40
Q_BIG = 160
Q_SMALL = 152
N_BIG = 17


def _prenorm_body(feat_ref, onorm_ref, o_ref):
    o_ref[...] = feat_ref[...] / onorm_ref[...]


def _postnorm_matmul_body(parts_ref, inorm_ref, w_ref, b_ref, o_ref):
    n = o_ref.shape[0]
    h2 = (parts_ref[0, :n, :] + parts_ref[1, :n, :]) / inorm_ref[...]
    o_ref[...] = jax.lax.dot_general(
        h2, w_ref[...], (((1,), (1,)), ((), ())),
        preferred_element_type=jnp.float32) + b_ref[...]


def _edge_agg_body(src_hbm, dst_hbm, hp_hbm, zero_hbm,
                   part_hbm, src_v, dst_v, rows_v, agg_sh, *sems):
    c = lax.axis_index("c")
    s = lax.axis_index("s")
    w = c * NS + s

    t0 = s * ROWS_PER_TILE
    pltpu.sync_copy(zero_hbm, agg_sh.at[pl.ds(t0, ROWS_PER_TILE)])

    def pipeline(start, q):
        pltpu.sync_copy(src_hbm.at[pl.ds(start, q)], src_v.at[pl.ds(0, q)])
        pltpu.sync_copy(dst_hbm.at[pl.ds(start, q)], dst_v.at[pl.ds(0, q)])

        for b in range(NBUF):
            pltpu.async_copy(hp_hbm.at[src_v.at[b]], rows_v.at[b], sems[b])

        def body(i, carry):
            j = i * NBUF
            for b in range(NBUF):
                m = j + b
                pltpu.make_async_copy(hp_hbm.at[src_v.at[m]], rows_v.at[b],
                                      sems[b]).wait()
                pltpu.sync_copy(rows_v.at[b], agg_sh.at[dst_v.at[m]],
                                add=True)
                nm = m + NBUF

                @pl.when(nm < q)
                def _():
                    pltpu.async_copy(hp_hbm.at[src_v.at[nm]], rows_v.at[b],
                                     sems[b])
            return carry

        lax.fori_loop(0, q // NBUF, body, 0)

    def process(start, q_total):
        for p in range(0, q_total, PHASE):
            pipeline(start + p, min(PHASE, q_total - p))

    @pl.when(w < N_BIG)
    def _():
        process(w * Q_BIG, Q_BIG)

    @pl.when(w >= N_BIG)
    def _():
        process(N_BIG * Q_BIG + (w - N_BIG) * Q_SMALL, Q_SMALL)

    plsc.subcore_barrier()
    pltpu.sync_copy(agg_sh.at[pl.ds(t0, ROWS_PER_TILE)],
                    part_hbm.at[c, pl.ds(t0, ROWS_PER_TILE)])


def _edge_aggregate(src2, dst2, hp, zero):
    mesh = plsc.VectorSubcoreMesh(core_axis_name="c", subcore_axis_name="s")
    return pl.kernel(
        _edge_agg_body,
        out_type=jax.ShapeDtypeStruct((NC, N_PAD, 128), jnp.float32),
        mesh=mesh,
        scratch_types=[
            pltpu.VMEM((PHASE, CHUNK), jnp.int32),
            pltpu.VMEM((PHASE, CHUNK), jnp.int32),
            pltpu.VMEM((NBUF, CHUNK, 128), jnp.float32),
            pltpu.VMEM_SHARED((N_PAD, 128), jnp.float32),
        ] + [pltpu.SemaphoreType.DMA] * NBUF,
    )(src2, dst2, hp, zero)


@jax.jit
def kernel(feat, edge_index, in_norm, out_norm, W, b):
    n, d_in = feat.shape
    e = edge_index.shape[1]

    src2 = edge_index[0].reshape(-1, CHUNK)
    dst2 = edge_index[1].reshape(-1, CHUNK)

    zero = jnp.zeros((ROWS_PER_TILE, 128), jnp.float32)

    hp = pl.pallas_call(
        _prenorm_body,
        out_shape=jax.ShapeDtypeStruct((n, 128), jnp.float32),
    )(feat, out_norm[:, None])

    parts = _edge_aggregate(src2, dst2, hp, zero)

    out = pl.pallas_call(
        _postnorm_matmul_body,
        out_shape=jax.ShapeDtypeStruct((n, 128), jnp.float32),
    )(parts, in_norm[:, None], W, b[None, :])

    return out
